# baseline (speedup 1.0000x reference)
import functools

import jax
import jax.numpy as jnp
import numpy as np
from jax import lax
from jax.experimental import pallas as pl
from jax.experimental.pallas import tpu as pltpu
from jax.experimental.pallas import tpu_sc as plsc

B, S, D, H, DFF, E = 1, 2048, 768, 12, 2048, 8
DH = D // H
QCH = 128

P2 = 2 * S
RB = 256
NBLK_MAX = P2 // RB + E
PMAX = NBLK_MAX * RB

NC, NS = 2, 16
NW = NC * NS
PPW = P2 // NW
CH = 64

_INV_SQRT2 = np.float32(1.0 / np.sqrt(2.0))
_INV_SQRT_DH = np.float32(1.0 / np.sqrt(DH))
_NT = (((1,), (1,)), ((), ()))


def _gelu(x):
    return 0.5 * x * (1.0 + jax.lax.erf(x * _INV_SQRT2))


def _ntdot(a, b):
    return jax.lax.dot_general(a, b, _NT, preferred_element_type=jnp.float32)


def _qkv_kernel(x_ref, w_ref, b_ref, qkv_ref):
    qkv_ref[...] = _ntdot(x_ref[...], w_ref[...]) + b_ref[...]


def _attn_kernel(q_ref, k_ref, v_ref, o_ref):
    for sub in range(2):
        q = q_ref[:, sub * DH:(sub + 1) * DH]
        k = k_ref[:, sub * DH:(sub + 1) * DH]
        v = v_ref[:, sub * DH:(sub + 1) * DH]
        for c in range(S // QCH):
            qc = q[c * QCH:(c + 1) * QCH, :]
            s = _ntdot(qc, k) * _INV_SQRT_DH
            m = jnp.max(s, axis=-1, keepdims=True)
            p = jnp.exp(s - m)
            z = jnp.sum(p, axis=-1, keepdims=True)
            o_ref[sub, c * QCH:(c + 1) * QCH, :] = jnp.dot(
                p, v, preferred_element_type=jnp.float32) / z


def _router_kernel(o_ref, w_ref, b_ref, x_ref, g1_ref, b1_ref, gw_ref,
                   gb_ref, x1_ref, vals_ref, dest_ref, be_ref, nu_ref):
    a = b_ref[...]
    for hh in range(H):
        a = a + _ntdot(o_ref[hh], w_ref[:, hh * DH:(hh + 1) * DH])
    h = x_ref[...] + a
    m = jnp.mean(h, axis=-1, keepdims=True)
    c = h - m
    v = jnp.mean(c * c, axis=-1, keepdims=True)
    x1 = c * jax.lax.rsqrt(v + 1e-5) * g1_ref[...] + b1_ref[...]
    x1_ref[...] = x1

    logits = _ntdot(x1, gw_ref[...]) + gb_ref[...]
    lm = jnp.max(logits, axis=-1, keepdims=True)
    p = jnp.exp(logits - lm)
    p = p / jnp.sum(p, axis=-1, keepdims=True)

    iota = jax.lax.broadcasted_iota(jnp.int32, (S, E), 1)
    m0 = jnp.max(p, axis=-1, keepdims=True)
    i0 = jnp.min(jnp.where(p == m0, iota, E), axis=-1, keepdims=True)
    oh0 = (iota == i0)
    p2 = jnp.where(oh0, -jnp.inf, p)
    m1 = jnp.max(p2, axis=-1, keepdims=True)
    i1 = jnp.min(jnp.where(p2 == m1, iota, E), axis=-1, keepdims=True)
    oh1 = (iota == i1)
    vals_ref[...] = jnp.concatenate([m0, m1], axis=1)

    ohp = jnp.concatenate([oh0.astype(jnp.float32), oh1.astype(jnp.float32)],
                          axis=0)
    inc = ohp
    sh = 1
    while sh < P2:
        inc = inc + jnp.concatenate(
            [jnp.zeros((sh, E), jnp.float32), inc[:P2 - sh, :]], axis=0)
        sh *= 2
    counts = inc[P2 - 1:P2, :]
    nb = jnp.maximum(jnp.floor((counts + np.float32(RB - 1))
                               * np.float32(1.0 / RB)), 1.0)
    ui = jax.lax.broadcasted_iota(jnp.int32, (E, E), 0)
    uj = jax.lax.broadcasted_iota(jnp.int32, (E, E), 1)
    upper = (ui < uj).astype(jnp.float32)
    excl = jnp.dot(nb, upper, preferred_element_type=jnp.float32)
    destf = jnp.sum(ohp * (np.float32(RB) * excl + inc - 1.0),
                    axis=1, keepdims=True)
    dest_ref[...] = destf.astype(jnp.int32)

    end = excl + nb
    ji = jax.lax.broadcasted_iota(
        jnp.int32, (NBLK_MAX, E), 0).astype(jnp.float32)
    be = jnp.sum((ji >= end).astype(jnp.float32), axis=1, keepdims=True)
    be_ref[...] = jnp.minimum(be, np.float32(E - 1)).astype(jnp.int32)
    nu_ref[...] = jnp.sum(nb, axis=1, keepdims=True).astype(jnp.int32)


@functools.lru_cache(maxsize=None)
def _sc_kernels():
    mesh = plsc.VectorSubcoreMesh(core_axis_name="c", subcore_axis_name="s")
    scratch = [
        pltpu.VMEM((CH,), jnp.int32),
        pltpu.VMEM((CH, D), jnp.float32),
        pltpu.SemaphoreType.DMA,
    ]

    @functools.partial(
        pl.kernel, mesh=mesh,
        out_type=jax.ShapeDtypeStruct((PMAX, D), jnp.float32),
        scratch_types=scratch,
    )
    def dispatch(x1_hbm, dest_hbm, gath_hbm, idx_v, rows_v, sem):
        wid = lax.axis_index("s") * NC + lax.axis_index("c")
        base = wid * PPW
        for cc in range(PPW // CH):
            off = base + cc * CH
            pltpu.sync_copy(dest_hbm.at[pl.ds(off, CH)], idx_v)
            tok = jnp.where(off >= S, off - S, off)
            pltpu.sync_copy(x1_hbm.at[pl.ds(tok, CH)], rows_v)
            pltpu.async_copy(rows_v, gath_hbm.at[idx_v], sem).wait()

    @functools.partial(
        pl.kernel, mesh=mesh,
        out_type=jax.ShapeDtypeStruct((P2, D), jnp.float32),
        scratch_types=scratch,
    )
    def collect(y_hbm, dest_hbm, y01_hbm, idx_v, rows_v, sem):
        wid = lax.axis_index("s") * NC + lax.axis_index("c")
        base = wid * PPW
        for cc in range(PPW // CH):
            off = base + cc * CH
            pltpu.sync_copy(dest_hbm.at[pl.ds(off, CH)], idx_v)
            pltpu.async_copy(y_hbm.at[idx_v], rows_v, sem).wait()
            pltpu.sync_copy(rows_v, y01_hbm.at[pl.ds(off, CH)])

    return dispatch, collect


def _sc_dispatch(x1, dest1):
    return _sc_kernels()[0](x1, dest1)


def _sc_collect(y, dest1):
    return _sc_kernels()[1](y, dest1)


def _ffn_kernel(meta_ref, g_ref, w1_ref, b1_ref, w2_ref, b2_ref,
                y_ref, base_ref):
    j = pl.program_id(0)

    @pl.when(j < meta_ref[NBLK_MAX])
    def _():
        b1v = b1_ref[0]
        gb1 = _gelu(b1v)
        h = _gelu(_ntdot(g_ref[...], w1_ref[0]) + b1v) - gb1
        w2v = w2_ref[0]
        y_ref[...] = _ntdot(h, w2v)
        base_ref[0] = _ntdot(gb1, w2v) + b2_ref[0]


def _combine_kernel(x1_ref, y01_ref, vals_ref, base_ref, g2_ref, be2_ref,
                    out_ref):
    v0 = vals_ref[:, 0:1]
    v1 = vals_ref[:, 1:2]
    bsum = jnp.sum(base_ref[:, 0, :], axis=0, keepdims=True)
    z = (x1_ref[...] + v0 * y01_ref[0] + v1 * y01_ref[1] + (v0 + v1) * bsum)
    m = jnp.mean(z, axis=-1, keepdims=True)
    c = z - m
    v = jnp.mean(c * c, axis=-1, keepdims=True)
    out_ref[...] = c * jax.lax.rsqrt(v + 1e-5) * g2_ref[...] + be2_ref[...]


def kernel(x, in_proj_w, in_proj_b, out_proj_w, out_proj_b, ln1_g, ln1_b,
           ln2_g, ln2_b, gate_w, gate_b, w1, b1, w2, b2):
    f32 = jnp.float32
    xs = x.reshape(S, D)

    qkv = pl.pallas_call(
        _qkv_kernel,
        in_specs=[
            pl.BlockSpec((S, D), lambda: (0, 0)),
            pl.BlockSpec((3 * D, D), lambda: (0, 0)),
            pl.BlockSpec((1, 3 * D), lambda: (0, 0)),
        ],
        out_specs=pl.BlockSpec((S, 3 * D), lambda: (0, 0)),
        out_shape=jax.ShapeDtypeStruct((S, 3 * D), f32),
    )(xs, in_proj_w, in_proj_b.reshape(1, 3 * D))

    H2 = H // 2
    attn_o = pl.pallas_call(
        _attn_kernel,
        grid=(H2,),
        in_specs=[
            pl.BlockSpec((S, 2 * DH), lambda i: (0, i)),
            pl.BlockSpec((S, 2 * DH), lambda i: (0, H2 + i)),
            pl.BlockSpec((S, 2 * DH), lambda i: (0, 2 * H2 + i)),
        ],
        out_specs=pl.BlockSpec((2, S, DH), lambda i: (i, 0, 0)),
        out_shape=jax.ShapeDtypeStruct((H, S, DH), f32),
    )(qkv, qkv, qkv)

    x1, vals, dest, be, nu = pl.pallas_call(
        _router_kernel,
        in_specs=[
            pl.BlockSpec((H, S, DH), lambda: (0, 0, 0)),
            pl.BlockSpec((D, D), lambda: (0, 0)),
            pl.BlockSpec((1, D), lambda: (0, 0)),
            pl.BlockSpec((S, D), lambda: (0, 0)),
            pl.BlockSpec((1, D), lambda: (0, 0)),
            pl.BlockSpec((1, D), lambda: (0, 0)),
            pl.BlockSpec((E, D), lambda: (0, 0)),
            pl.BlockSpec((1, E), lambda: (0, 0)),
        ],
        out_specs=[
            pl.BlockSpec((S, D), lambda: (0, 0)),
            pl.BlockSpec((S, 2), lambda: (0, 0)),
            pl.BlockSpec((P2, 1), lambda: (0, 0)),
            pl.BlockSpec((NBLK_MAX, 1), lambda: (0, 0)),
            pl.BlockSpec((1, 1), lambda: (0, 0)),
        ],
        out_shape=[
            jax.ShapeDtypeStruct((S, D), f32),
            jax.ShapeDtypeStruct((S, 2), f32),
            jax.ShapeDtypeStruct((P2, 1), jnp.int32),
            jax.ShapeDtypeStruct((NBLK_MAX, 1), jnp.int32),
            jax.ShapeDtypeStruct((1, 1), jnp.int32),
        ],
    )(attn_o, out_proj_w, out_proj_b.reshape(1, D), xs,
      ln1_g.reshape(1, D), ln1_b.reshape(1, D), gate_w,
      gate_b.reshape(1, E))

    dest1 = dest.reshape(P2)
    meta = jnp.concatenate([be.reshape(NBLK_MAX), nu.reshape(1)])

    gathered = _sc_dispatch(x1, dest1)

    b1r = b1.reshape(E, 1, DFF)
    b2r = b2.reshape(E, 1, D)
    y, base_mat = pl.pallas_call(
        _ffn_kernel,
        grid_spec=pltpu.PrefetchScalarGridSpec(
            num_scalar_prefetch=1,
            grid=(NBLK_MAX,),
            in_specs=[
                pl.BlockSpec((RB, D), lambda j, meta: (j, 0)),
                pl.BlockSpec((1, DFF, D), lambda j, meta: (meta[j], 0, 0)),
                pl.BlockSpec((1, 1, DFF), lambda j, meta: (meta[j], 0, 0)),
                pl.BlockSpec((1, D, DFF), lambda j, meta: (meta[j], 0, 0)),
                pl.BlockSpec((1, 1, D), lambda j, meta: (meta[j], 0, 0)),
            ],
            out_specs=[
                pl.BlockSpec((RB, D), lambda j, meta: (j, 0)),
                pl.BlockSpec((1, 1, D), lambda j, meta: (meta[j], 0, 0)),
            ],
        ),
        out_shape=[
            jax.ShapeDtypeStruct((PMAX, D), f32),
            jax.ShapeDtypeStruct((E, 1, D), f32),
        ],
    )(meta, gathered, w1, b1r, w2, b2r)

    y01 = _sc_collect(y, dest1).reshape(2, S, D)

    out = pl.pallas_call(
        _combine_kernel,
        in_specs=[
            pl.BlockSpec((S, D), lambda: (0, 0)),
            pl.BlockSpec((2, S, D), lambda: (0, 0, 0)),
            pl.BlockSpec((S, 2), lambda: (0, 0)),
            pl.BlockSpec((E, 1, D), lambda: (0, 0, 0)),
            pl.BlockSpec((1, D), lambda: (0, 0)),
            pl.BlockSpec((1, D), lambda: (0, 0)),
        ],
        out_specs=pl.BlockSpec((S, D), lambda: (0, 0)),
        out_shape=jax.ShapeDtypeStruct((S, D), f32),
    )(x1, y01, vals, base_mat, ln2_g.reshape(1, D), ln2_b.reshape(1, D))

    return out.reshape(B, S, D)


# speedup vs baseline: 1.0178x; 1.0178x over previous
import functools

import jax
import jax.numpy as jnp
import numpy as np
from jax import lax
from jax.experimental import pallas as pl
from jax.experimental.pallas import tpu as pltpu
from jax.experimental.pallas import tpu_sc as plsc

B, S, D, H, DFF, E = 1, 2048, 768, 12, 2048, 8
DH = D // H
QCH = 256
DFF2 = DFF // 2

P2 = 2 * S
RB = 256
NBLK_MAX = P2 // RB + E
PMAX = NBLK_MAX * RB

NC, NS = 2, 16
NW = NC * NS
PPW = P2 // NW
CH = 64

_INV_SQRT2 = np.float32(1.0 / np.sqrt(2.0))
_INV_SQRT_DH = np.float32(1.0 / np.sqrt(DH))
_NT = (((1,), (1,)), ((), ()))


def _gelu(x):
    return 0.5 * x * (1.0 + jax.lax.erf(x * _INV_SQRT2))


def _ntdot(a, b):
    return jax.lax.dot_general(a, b, _NT, preferred_element_type=jnp.float32)


def _qkv_kernel(x_ref, w_ref, b_ref, qkv_ref):
    qkv_ref[...] = _ntdot(x_ref[...], w_ref[...]) + b_ref[...]


def _attn_kernel(q_ref, k_ref, v_ref, o_ref):
    for sub in range(2):
        q = q_ref[:, sub * DH:(sub + 1) * DH]
        k = k_ref[:, sub * DH:(sub + 1) * DH]
        v = v_ref[:, sub * DH:(sub + 1) * DH]
        for c in range(S // QCH):
            qc = q[c * QCH:(c + 1) * QCH, :]
            s = _ntdot(qc, k) * _INV_SQRT_DH
            m = jnp.max(s, axis=-1, keepdims=True)
            p = jnp.exp(s - m)
            z = jnp.sum(p, axis=-1, keepdims=True)
            o_ref[sub, c * QCH:(c + 1) * QCH, :] = jnp.dot(
                p, v, preferred_element_type=jnp.float32) / z


def _router_kernel(o_ref, w_ref, b_ref, x_ref, g1_ref, b1_ref, gw_ref,
                   gb_ref, x1_ref, vals_ref, dest_ref, be_ref, nu_ref):
    a = b_ref[...]
    for hh in range(H):
        a = a + _ntdot(o_ref[hh], w_ref[:, hh * DH:(hh + 1) * DH])
    h = x_ref[...] + a
    m = jnp.mean(h, axis=-1, keepdims=True)
    c = h - m
    v = jnp.mean(c * c, axis=-1, keepdims=True)
    x1 = c * jax.lax.rsqrt(v + 1e-5) * g1_ref[...] + b1_ref[...]
    x1_ref[...] = x1

    logits = _ntdot(x1, gw_ref[...]) + gb_ref[...]
    lm = jnp.max(logits, axis=-1, keepdims=True)
    p = jnp.exp(logits - lm)
    p = p / jnp.sum(p, axis=-1, keepdims=True)

    iota = jax.lax.broadcasted_iota(jnp.int32, (S, E), 1)
    m0 = jnp.max(p, axis=-1, keepdims=True)
    i0 = jnp.min(jnp.where(p == m0, iota, E), axis=-1, keepdims=True)
    oh0 = (iota == i0)
    p2 = jnp.where(oh0, -jnp.inf, p)
    m1 = jnp.max(p2, axis=-1, keepdims=True)
    i1 = jnp.min(jnp.where(p2 == m1, iota, E), axis=-1, keepdims=True)
    oh1 = (iota == i1)
    vals_ref[...] = jnp.concatenate([m0, m1], axis=1)

    ohp = jnp.concatenate([oh0.astype(jnp.float32), oh1.astype(jnp.float32)],
                          axis=0)
    inc = ohp
    sh = 1
    while sh < P2:
        inc = inc + jnp.concatenate(
            [jnp.zeros((sh, E), jnp.float32), inc[:P2 - sh, :]], axis=0)
        sh *= 2
    counts = inc[P2 - 1:P2, :]
    nb = jnp.maximum(jnp.floor((counts + np.float32(RB - 1))
                               * np.float32(1.0 / RB)), 1.0)
    ui = jax.lax.broadcasted_iota(jnp.int32, (E, E), 0)
    uj = jax.lax.broadcasted_iota(jnp.int32, (E, E), 1)
    upper = (ui < uj).astype(jnp.float32)
    excl = jnp.dot(nb, upper, preferred_element_type=jnp.float32)
    destf = jnp.sum(ohp * (np.float32(RB) * excl + inc - 1.0),
                    axis=1, keepdims=True)
    dest_ref[...] = destf.astype(jnp.int32)

    end = excl + nb
    ji = jax.lax.broadcasted_iota(
        jnp.int32, (NBLK_MAX, E), 0).astype(jnp.float32)
    be = jnp.sum((ji >= end).astype(jnp.float32), axis=1, keepdims=True)
    be_ref[...] = jnp.minimum(be, np.float32(E - 1)).astype(jnp.int32)
    nu_ref[...] = jnp.sum(nb, axis=1, keepdims=True).astype(jnp.int32)


@functools.lru_cache(maxsize=None)
def _sc_kernels():
    mesh = plsc.VectorSubcoreMesh(core_axis_name="c", subcore_axis_name="s")
    scratch = [
        pltpu.VMEM((CH,), jnp.int32),
        pltpu.VMEM((CH, D), jnp.float32),
        pltpu.SemaphoreType.DMA,
    ]

    @functools.partial(
        pl.kernel, mesh=mesh,
        out_type=jax.ShapeDtypeStruct((PMAX, D), jnp.float32),
        scratch_types=scratch,
    )
    def dispatch(x1_hbm, dest_hbm, gath_hbm, idx_v, rows_v, sem):
        wid = lax.axis_index("s") * NC + lax.axis_index("c")
        base = wid * PPW
        for cc in range(PPW // CH):
            off = base + cc * CH
            pltpu.sync_copy(dest_hbm.at[pl.ds(off, CH)], idx_v)
            tok = jnp.where(off >= S, off - S, off)
            pltpu.sync_copy(x1_hbm.at[pl.ds(tok, CH)], rows_v)
            pltpu.async_copy(rows_v, gath_hbm.at[idx_v], sem).wait()

    @functools.partial(
        pl.kernel, mesh=mesh,
        out_type=jax.ShapeDtypeStruct((P2, D), jnp.float32),
        scratch_types=scratch,
    )
    def collect(y_hbm, dest_hbm, y01_hbm, idx_v, rows_v, sem):
        wid = lax.axis_index("s") * NC + lax.axis_index("c")
        base = wid * PPW
        for cc in range(PPW // CH):
            off = base + cc * CH
            pltpu.sync_copy(dest_hbm.at[pl.ds(off, CH)], idx_v)
            pltpu.async_copy(y_hbm.at[idx_v], rows_v, sem).wait()
            pltpu.sync_copy(rows_v, y01_hbm.at[pl.ds(off, CH)])

    return dispatch, collect


def _sc_dispatch(x1, dest1):
    return _sc_kernels()[0](x1, dest1)


def _sc_collect(y, dest1):
    return _sc_kernels()[1](y, dest1)


def _ffn_kernel(meta_ref, g_ref, w1_ref, b1_ref, w2_ref, b2_ref,
                y_ref, base_ref, acc_ref, accb_ref):
    j = pl.program_id(0)
    g = pl.program_id(1)

    @pl.when(j < meta_ref[NBLK_MAX])
    def _():
        b1v = b1_ref[0]
        gb1 = _gelu(b1v)
        h = _gelu(_ntdot(g_ref[...], w1_ref[0]) + b1v) - gb1
        w2v = w2_ref[0]
        yh = _ntdot(h, w2v)
        bh = _ntdot(gb1, w2v)

        @pl.when(g == 0)
        def _g0():
            acc_ref[...] = yh
            accb_ref[...] = bh

        @pl.when(g == 1)
        def _g1():
            y_ref[...] = acc_ref[...] + yh
            base_ref[0] = accb_ref[...] + bh + b2_ref[0]


def _combine_kernel(x1_ref, y01_ref, vals_ref, base_ref, g2_ref, be2_ref,
                    out_ref):
    v0 = vals_ref[:, 0:1]
    v1 = vals_ref[:, 1:2]
    bsum = jnp.sum(base_ref[:, 0, :], axis=0, keepdims=True)
    z = (x1_ref[...] + v0 * y01_ref[0] + v1 * y01_ref[1] + (v0 + v1) * bsum)
    m = jnp.mean(z, axis=-1, keepdims=True)
    c = z - m
    v = jnp.mean(c * c, axis=-1, keepdims=True)
    out_ref[...] = c * jax.lax.rsqrt(v + 1e-5) * g2_ref[...] + be2_ref[...]


def kernel(x, in_proj_w, in_proj_b, out_proj_w, out_proj_b, ln1_g, ln1_b,
           ln2_g, ln2_b, gate_w, gate_b, w1, b1, w2, b2):
    f32 = jnp.float32
    xs = x.reshape(S, D)

    qkv = pl.pallas_call(
        _qkv_kernel,
        in_specs=[
            pl.BlockSpec((S, D), lambda: (0, 0)),
            pl.BlockSpec((3 * D, D), lambda: (0, 0)),
            pl.BlockSpec((1, 3 * D), lambda: (0, 0)),
        ],
        out_specs=pl.BlockSpec((S, 3 * D), lambda: (0, 0)),
        out_shape=jax.ShapeDtypeStruct((S, 3 * D), f32),
    )(xs, in_proj_w, in_proj_b.reshape(1, 3 * D))

    H2 = H // 2
    attn_o = pl.pallas_call(
        _attn_kernel,
        grid=(H2,),
        in_specs=[
            pl.BlockSpec((S, 2 * DH), lambda i: (0, i)),
            pl.BlockSpec((S, 2 * DH), lambda i: (0, H2 + i)),
            pl.BlockSpec((S, 2 * DH), lambda i: (0, 2 * H2 + i)),
        ],
        out_specs=pl.BlockSpec((2, S, DH), lambda i: (i, 0, 0)),
        out_shape=jax.ShapeDtypeStruct((H, S, DH), f32),
    )(qkv, qkv, qkv)

    x1, vals, dest, be, nu = pl.pallas_call(
        _router_kernel,
        in_specs=[
            pl.BlockSpec((H, S, DH), lambda: (0, 0, 0)),
            pl.BlockSpec((D, D), lambda: (0, 0)),
            pl.BlockSpec((1, D), lambda: (0, 0)),
            pl.BlockSpec((S, D), lambda: (0, 0)),
            pl.BlockSpec((1, D), lambda: (0, 0)),
            pl.BlockSpec((1, D), lambda: (0, 0)),
            pl.BlockSpec((E, D), lambda: (0, 0)),
            pl.BlockSpec((1, E), lambda: (0, 0)),
        ],
        out_specs=[
            pl.BlockSpec((S, D), lambda: (0, 0)),
            pl.BlockSpec((S, 2), lambda: (0, 0)),
            pl.BlockSpec((P2, 1), lambda: (0, 0)),
            pl.BlockSpec((NBLK_MAX, 1), lambda: (0, 0)),
            pl.BlockSpec((1, 1), lambda: (0, 0)),
        ],
        out_shape=[
            jax.ShapeDtypeStruct((S, D), f32),
            jax.ShapeDtypeStruct((S, 2), f32),
            jax.ShapeDtypeStruct((P2, 1), jnp.int32),
            jax.ShapeDtypeStruct((NBLK_MAX, 1), jnp.int32),
            jax.ShapeDtypeStruct((1, 1), jnp.int32),
        ],
    )(attn_o, out_proj_w, out_proj_b.reshape(1, D), xs,
      ln1_g.reshape(1, D), ln1_b.reshape(1, D), gate_w,
      gate_b.reshape(1, E))

    dest1 = dest.reshape(P2)
    meta = jnp.concatenate([be.reshape(NBLK_MAX), nu.reshape(1)])

    gathered = _sc_dispatch(x1, dest1)

    b1r = b1.reshape(E, 1, DFF)
    b2r = b2.reshape(E, 1, D)
    y, base_mat = pl.pallas_call(
        _ffn_kernel,
        grid_spec=pltpu.PrefetchScalarGridSpec(
            num_scalar_prefetch=1,
            grid=(NBLK_MAX, 2),
            in_specs=[
                pl.BlockSpec((RB, D), lambda j, g, meta: (j, 0)),
                pl.BlockSpec((1, DFF2, D),
                             lambda j, g, meta: (meta[j], g, 0)),
                pl.BlockSpec((1, 1, DFF2),
                             lambda j, g, meta: (meta[j], 0, g)),
                pl.BlockSpec((1, D, DFF2),
                             lambda j, g, meta: (meta[j], 0, g)),
                pl.BlockSpec((1, 1, D), lambda j, g, meta: (meta[j], 0, 0)),
            ],
            out_specs=[
                pl.BlockSpec((RB, D), lambda j, g, meta: (j, 0)),
                pl.BlockSpec((1, 1, D), lambda j, g, meta: (meta[j], 0, 0)),
            ],
            scratch_shapes=[
                pltpu.VMEM((RB, D), jnp.float32),
                pltpu.VMEM((1, D), jnp.float32),
            ],
        ),
        out_shape=[
            jax.ShapeDtypeStruct((PMAX, D), f32),
            jax.ShapeDtypeStruct((E, 1, D), f32),
        ],
    )(meta, gathered, w1, b1r, w2, b2r)

    y01 = _sc_collect(y, dest1).reshape(2, S, D)

    out = pl.pallas_call(
        _combine_kernel,
        in_specs=[
            pl.BlockSpec((S, D), lambda: (0, 0)),
            pl.BlockSpec((2, S, D), lambda: (0, 0, 0)),
            pl.BlockSpec((S, 2), lambda: (0, 0)),
            pl.BlockSpec((E, 1, D), lambda: (0, 0, 0)),
            pl.BlockSpec((1, D), lambda: (0, 0)),
            pl.BlockSpec((1, D), lambda: (0, 0)),
        ],
        out_specs=pl.BlockSpec((S, D), lambda: (0, 0)),
        out_shape=jax.ShapeDtypeStruct((S, D), f32),
    )(x1, y01, vals, base_mat, ln2_g.reshape(1, D), ln2_b.reshape(1, D))

    return out.reshape(B, S, D)


# speedup vs baseline: 1.1548x; 1.1347x over previous
import functools

import jax
import jax.numpy as jnp
import numpy as np
from jax import lax
from jax.experimental import pallas as pl
from jax.experimental.pallas import tpu as pltpu
from jax.experimental.pallas import tpu_sc as plsc

B, S, D, H, DFF, E = 1, 2048, 768, 12, 2048, 8
DH = D // H
QCH = 256
DFF2 = DFF // 2

P2 = 2 * S
RB = 256
NBLK_MAX = P2 // RB + E
PMAX = NBLK_MAX * RB

NC, NS = 2, 16
NW = NC * NS
PPW = P2 // NW
CH = 128

_INV_SQRT2 = np.float32(1.0 / np.sqrt(2.0))
_INV_SQRT_DH = np.float32(1.0 / np.sqrt(DH))
_NT = (((1,), (1,)), ((), ()))


def _gelu(x):
    return 0.5 * x * (1.0 + jax.lax.erf(x * _INV_SQRT2))


def _ntdot(a, b):
    return jax.lax.dot_general(a, b, _NT, preferred_element_type=jnp.float32)


def _qkv_kernel(x_ref, w_ref, b_ref, qkv_ref):
    qkv_ref[...] = _ntdot(x_ref[...], w_ref[...]) + b_ref[...]


def _attn_kernel(q_ref, k_ref, v_ref, o_ref):
    for sub in range(2):
        q = q_ref[:, sub * DH:(sub + 1) * DH]
        k = k_ref[:, sub * DH:(sub + 1) * DH]
        v = v_ref[:, sub * DH:(sub + 1) * DH]
        for c in range(S // QCH):
            qc = q[c * QCH:(c + 1) * QCH, :]
            s = _ntdot(qc, k) * _INV_SQRT_DH
            m = jnp.max(s, axis=-1, keepdims=True)
            p = jnp.exp(s - m)
            z = jnp.sum(p, axis=-1, keepdims=True)
            o_ref[sub, c * QCH:(c + 1) * QCH, :] = jnp.dot(
                p, v, preferred_element_type=jnp.float32) / z


def _router_kernel(o_ref, w_ref, b_ref, x_ref, g1_ref, b1_ref, gw_ref,
                   gb_ref, x1_ref, vals_ref, dest_ref, be_ref, nu_ref):
    a = b_ref[...]
    for hh in range(H):
        a = a + _ntdot(o_ref[hh], w_ref[:, hh * DH:(hh + 1) * DH])
    h = x_ref[...] + a
    m = jnp.mean(h, axis=-1, keepdims=True)
    c = h - m
    v = jnp.mean(c * c, axis=-1, keepdims=True)
    x1 = c * jax.lax.rsqrt(v + 1e-5) * g1_ref[...] + b1_ref[...]
    x1_ref[...] = x1

    logits = _ntdot(x1, gw_ref[...]) + gb_ref[...]
    lm = jnp.max(logits, axis=-1, keepdims=True)
    p = jnp.exp(logits - lm)
    p = p / jnp.sum(p, axis=-1, keepdims=True)

    iota = jax.lax.broadcasted_iota(jnp.int32, (S, E), 1)
    m0 = jnp.max(p, axis=-1, keepdims=True)
    i0 = jnp.min(jnp.where(p == m0, iota, E), axis=-1, keepdims=True)
    oh0 = (iota == i0)
    p2 = jnp.where(oh0, -jnp.inf, p)
    m1 = jnp.max(p2, axis=-1, keepdims=True)
    i1 = jnp.min(jnp.where(p2 == m1, iota, E), axis=-1, keepdims=True)
    oh1 = (iota == i1)
    vals_ref[...] = jnp.concatenate([m0, m1], axis=1)

    ohp = jnp.concatenate([oh0.astype(jnp.float32), oh1.astype(jnp.float32)],
                          axis=0)
    inc = ohp
    sh = 1
    while sh < P2:
        inc = inc + jnp.concatenate(
            [jnp.zeros((sh, E), jnp.float32), inc[:P2 - sh, :]], axis=0)
        sh *= 2
    counts = inc[P2 - 1:P2, :]
    nb = jnp.maximum(jnp.floor((counts + np.float32(RB - 1))
                               * np.float32(1.0 / RB)), 1.0)
    ui = jax.lax.broadcasted_iota(jnp.int32, (E, E), 0)
    uj = jax.lax.broadcasted_iota(jnp.int32, (E, E), 1)
    upper = (ui < uj).astype(jnp.float32)
    excl = jnp.dot(nb, upper, preferred_element_type=jnp.float32)
    destf = jnp.sum(ohp * (np.float32(RB) * excl + inc - 1.0),
                    axis=1, keepdims=True)
    dest_ref[...] = destf.astype(jnp.int32)

    end = excl + nb
    ji = jax.lax.broadcasted_iota(
        jnp.int32, (NBLK_MAX, E), 0).astype(jnp.float32)
    be = jnp.sum((ji >= end).astype(jnp.float32), axis=1, keepdims=True)
    be_ref[...] = jnp.minimum(be, np.float32(E - 1)).astype(jnp.int32)
    nu_ref[...] = jnp.sum(nb, axis=1, keepdims=True).astype(jnp.int32)


@functools.lru_cache(maxsize=None)
def _sc_kernels():
    mesh = plsc.VectorSubcoreMesh(core_axis_name="c", subcore_axis_name="s")
    scratch = [
        pltpu.VMEM((CH,), jnp.int32),
        pltpu.VMEM((CH, D), jnp.float32),
        pltpu.SemaphoreType.DMA,
    ]

    @functools.partial(
        pl.kernel, mesh=mesh,
        out_type=jax.ShapeDtypeStruct((PMAX, D), jnp.float32),
        scratch_types=scratch,
    )
    def dispatch(x1_hbm, dest_hbm, gath_hbm, idx_v, rows_v, sem):
        wid = lax.axis_index("s") * NC + lax.axis_index("c")
        base = wid * PPW
        for cc in range(PPW // CH):
            off = base + cc * CH
            pltpu.sync_copy(dest_hbm.at[pl.ds(off, CH)], idx_v)
            tok = jnp.where(off >= S, off - S, off)
            pltpu.sync_copy(x1_hbm.at[pl.ds(tok, CH)], rows_v)
            pltpu.async_copy(rows_v, gath_hbm.at[idx_v], sem).wait()

    @functools.partial(
        pl.kernel, mesh=mesh,
        out_type=jax.ShapeDtypeStruct((P2, D), jnp.float32),
        scratch_types=scratch,
    )
    def collect(y_hbm, dest_hbm, y01_hbm, idx_v, rows_v, sem):
        wid = lax.axis_index("s") * NC + lax.axis_index("c")
        base = wid * PPW
        for cc in range(PPW // CH):
            off = base + cc * CH
            pltpu.sync_copy(dest_hbm.at[pl.ds(off, CH)], idx_v)
            pltpu.async_copy(y_hbm.at[idx_v], rows_v, sem).wait()
            pltpu.sync_copy(rows_v, y01_hbm.at[pl.ds(off, CH)])

    return dispatch, collect


def _sc_dispatch(x1, dest1):
    return _sc_kernels()[0](x1, dest1)


def _sc_collect(y, dest1):
    return _sc_kernels()[1](y, dest1)


def _ffn_kernel(meta_ref, g_ref, w1_ref, b1_ref, w2_ref, b2_ref,
                y_ref, base_ref):
    j = pl.program_id(0)

    @pl.when(j < meta_ref[NBLK_MAX])
    def _():
        b1v = b1_ref[0]
        gb1 = _gelu(b1v)
        h = _gelu(_ntdot(g_ref[...], w1_ref[0]) + b1v) - gb1
        w2v = w2_ref[0]
        y_ref[...] = _ntdot(h, w2v)
        base_ref[0] = _ntdot(gb1, w2v) + b2_ref[0]


def _combine_kernel(x1_ref, y01_ref, vals_ref, base_ref, g2_ref, be2_ref,
                    out_ref):
    v0 = vals_ref[:, 0:1]
    v1 = vals_ref[:, 1:2]
    bsum = jnp.sum(base_ref[:, 0, :], axis=0, keepdims=True)
    z = (x1_ref[...] + v0 * y01_ref[0] + v1 * y01_ref[1] + (v0 + v1) * bsum)
    m = jnp.mean(z, axis=-1, keepdims=True)
    c = z - m
    v = jnp.mean(c * c, axis=-1, keepdims=True)
    out_ref[...] = c * jax.lax.rsqrt(v + 1e-5) * g2_ref[...] + be2_ref[...]


def kernel(x, in_proj_w, in_proj_b, out_proj_w, out_proj_b, ln1_g, ln1_b,
           ln2_g, ln2_b, gate_w, gate_b, w1, b1, w2, b2):
    f32 = jnp.float32
    xs = x.reshape(S, D)

    qkv = pl.pallas_call(
        _qkv_kernel,
        in_specs=[
            pl.BlockSpec((S, D), lambda: (0, 0)),
            pl.BlockSpec((3 * D, D), lambda: (0, 0)),
            pl.BlockSpec((1, 3 * D), lambda: (0, 0)),
        ],
        out_specs=pl.BlockSpec((S, 3 * D), lambda: (0, 0)),
        out_shape=jax.ShapeDtypeStruct((S, 3 * D), f32),
    )(xs, in_proj_w, in_proj_b.reshape(1, 3 * D))

    H2 = H // 2
    attn_o = pl.pallas_call(
        _attn_kernel,
        grid=(H2,),
        in_specs=[
            pl.BlockSpec((S, 2 * DH), lambda i: (0, i)),
            pl.BlockSpec((S, 2 * DH), lambda i: (0, H2 + i)),
            pl.BlockSpec((S, 2 * DH), lambda i: (0, 2 * H2 + i)),
        ],
        out_specs=pl.BlockSpec((2, S, DH), lambda i: (i, 0, 0)),
        out_shape=jax.ShapeDtypeStruct((H, S, DH), f32),
    )(qkv, qkv, qkv)

    x1, vals, dest, be, nu = pl.pallas_call(
        _router_kernel,
        in_specs=[
            pl.BlockSpec((H, S, DH), lambda: (0, 0, 0)),
            pl.BlockSpec((D, D), lambda: (0, 0)),
            pl.BlockSpec((1, D), lambda: (0, 0)),
            pl.BlockSpec((S, D), lambda: (0, 0)),
            pl.BlockSpec((1, D), lambda: (0, 0)),
            pl.BlockSpec((1, D), lambda: (0, 0)),
            pl.BlockSpec((E, D), lambda: (0, 0)),
            pl.BlockSpec((1, E), lambda: (0, 0)),
        ],
        out_specs=[
            pl.BlockSpec((S, D), lambda: (0, 0)),
            pl.BlockSpec((S, 2), lambda: (0, 0)),
            pl.BlockSpec((P2, 1), lambda: (0, 0)),
            pl.BlockSpec((NBLK_MAX, 1), lambda: (0, 0)),
            pl.BlockSpec((1, 1), lambda: (0, 0)),
        ],
        out_shape=[
            jax.ShapeDtypeStruct((S, D), f32),
            jax.ShapeDtypeStruct((S, 2), f32),
            jax.ShapeDtypeStruct((P2, 1), jnp.int32),
            jax.ShapeDtypeStruct((NBLK_MAX, 1), jnp.int32),
            jax.ShapeDtypeStruct((1, 1), jnp.int32),
        ],
    )(attn_o, out_proj_w, out_proj_b.reshape(1, D), xs,
      ln1_g.reshape(1, D), ln1_b.reshape(1, D), gate_w,
      gate_b.reshape(1, E))

    dest1 = dest.reshape(P2)
    meta = jnp.concatenate([be.reshape(NBLK_MAX), nu.reshape(1)])

    gathered = _sc_dispatch(x1, dest1)

    b1r = b1.reshape(E, 1, DFF)
    b2r = b2.reshape(E, 1, D)
    y, base_mat = pl.pallas_call(
        _ffn_kernel,
        grid_spec=pltpu.PrefetchScalarGridSpec(
            num_scalar_prefetch=1,
            grid=(NBLK_MAX,),
            in_specs=[
                pl.BlockSpec((RB, D), lambda j, meta: (j, 0)),
                pl.BlockSpec((1, DFF, D), lambda j, meta: (meta[j], 0, 0)),
                pl.BlockSpec((1, 1, DFF), lambda j, meta: (meta[j], 0, 0)),
                pl.BlockSpec((1, D, DFF), lambda j, meta: (meta[j], 0, 0)),
                pl.BlockSpec((1, 1, D), lambda j, meta: (meta[j], 0, 0)),
            ],
            out_specs=[
                pl.BlockSpec((RB, D), lambda j, meta: (j, 0)),
                pl.BlockSpec((1, 1, D), lambda j, meta: (meta[j], 0, 0)),
            ],
        ),
        out_shape=[
            jax.ShapeDtypeStruct((PMAX, D), f32),
            jax.ShapeDtypeStruct((E, 1, D), f32),
        ],
    )(meta, gathered, w1, b1r, w2, b2r)

    y01 = _sc_collect(y, dest1).reshape(2, S, D)

    out = pl.pallas_call(
        _combine_kernel,
        in_specs=[
            pl.BlockSpec((S, D), lambda: (0, 0)),
            pl.BlockSpec((2, S, D), lambda: (0, 0, 0)),
            pl.BlockSpec((S, 2), lambda: (0, 0)),
            pl.BlockSpec((E, 1, D), lambda: (0, 0, 0)),
            pl.BlockSpec((1, D), lambda: (0, 0)),
            pl.BlockSpec((1, D), lambda: (0, 0)),
        ],
        out_specs=pl.BlockSpec((S, D), lambda: (0, 0)),
        out_shape=jax.ShapeDtypeStruct((S, D), f32),
    )(x1, y01, vals, base_mat, ln2_g.reshape(1, D), ln2_b.reshape(1, D))

    return out.reshape(B, S, D)
